# Initial kernel scaffold; baseline (speedup 1.0000x reference)
import jax
import jax.numpy as jnp
from jax.experimental import pallas as pl


def kernel(x, edge_index, W, b):
    raise NotImplementedError("write your pallas kernel here")


# speedup vs baseline: 8.4440x; 8.4440x over previous
import functools

import jax
import jax.numpy as jnp
from jax import lax
from jax.experimental import pallas as pl
from jax.experimental.pallas import tpu as pltpu
from jax.experimental.pallas import tpu_sc as plsc

NC = 2
NS = 16
L = 16
CHUNK = 128


def _deg_kernel(n_pad, e_pad):
    n_per_tile = n_pad // NS
    e_per_w = e_pad // (NC * NS)
    n_chunks = e_per_w // CHUNK
    mesh = plsc.VectorSubcoreMesh(core_axis_name="c", subcore_axis_name="s")

    @functools.partial(
        pl.kernel,
        out_type=jax.ShapeDtypeStruct((NC, n_pad, L), jnp.float32),
        mesh=mesh,
        scratch_types=[
            pltpu.VMEM_SHARED((n_pad, L), jnp.float32),
            pltpu.VMEM((n_per_tile, L), jnp.float32),
            pltpu.VMEM((CHUNK, L), jnp.float32),
            pltpu.VMEM((CHUNK,), jnp.int32),
        ],
    )
    def k(dst_hbm, out_hbm, acc_sh, zbuf, ones_v, idx_v):
        c = lax.axis_index("c")
        s = lax.axis_index("s")
        wid = c * NS + s

        def fill(i, _):
            zbuf[i, :] = jnp.zeros((L,), jnp.float32)
            return 0

        lax.fori_loop(0, n_per_tile, fill, 0)

        def fill1(i, _):
            ones_v[i, :] = jnp.full((L,), 1.0, jnp.float32)
            return 0

        lax.fori_loop(0, CHUNK, fill1, 0)
        pltpu.sync_copy(zbuf, acc_sh.at[pl.ds(s * n_per_tile, n_per_tile)])
        plsc.subcore_barrier()

        base = wid * e_per_w

        def body(kk, _):
            pltpu.sync_copy(dst_hbm.at[pl.ds(base + kk * CHUNK, CHUNK)], idx_v)
            pltpu.sync_copy(ones_v, acc_sh.at[idx_v], add=True)
            return 0

        lax.fori_loop(0, n_chunks, body, 0)
        plsc.subcore_barrier()
        pltpu.sync_copy(
            acc_sh.at[pl.ds(s * n_per_tile, n_per_tile)],
            out_hbm.at[c].at[pl.ds(s * n_per_tile, n_per_tile)],
        )

    return k


def _edge_kernel(n_pad, e_pad, dh):
    n_per_tile = n_pad // NS
    e_per_tile = e_pad // NS
    n_chunks = e_per_tile // CHUNK
    zrows = 128
    mesh = plsc.VectorSubcoreMesh(core_axis_name="c", subcore_axis_name="s")

    @functools.partial(
        pl.kernel,
        out_type=jax.ShapeDtypeStruct((NC, n_pad, dh), jnp.float32),
        mesh=mesh,
        scratch_types=[
            pltpu.VMEM_SHARED((n_pad, dh), jnp.float32),
            pltpu.VMEM((zrows, dh), jnp.float32),
            pltpu.VMEM((CHUNK, dh), jnp.float32),
            pltpu.VMEM((CHUNK,), jnp.int32),
            pltpu.VMEM((CHUNK,), jnp.int32),
            pltpu.SemaphoreType.DMA,
        ],
    )
    def k(src_hbm, dst_hbm, hs_hbm, out_hbm, acc_sh, zbuf, rows_v, isrc, idst, sem):
        c = lax.axis_index("c")
        s = lax.axis_index("s")

        def fill(i, _):
            for j in range(dh // L):
                zbuf[i, pl.ds(j * L, L)] = jnp.zeros((L,), jnp.float32)
            return 0

        lax.fori_loop(0, zrows, fill, 0)
        for r in range(n_per_tile // zrows):
            pltpu.sync_copy(
                zbuf, acc_sh.at[pl.ds(s * n_per_tile + r * zrows, zrows)]
            )
        plsc.subcore_barrier()

        base = s * e_per_tile

        def body(kk, _):
            off = base + kk * CHUNK
            pltpu.sync_copy(src_hbm.at[pl.ds(off, CHUNK)], isrc)
            pltpu.sync_copy(dst_hbm.at[pl.ds(off, CHUNK)], idst)
            pltpu.async_copy(hs_hbm.at[c].at[isrc], rows_v, sem).wait()
            pltpu.sync_copy(rows_v, acc_sh.at[idst], add=True)
            return 0

        lax.fori_loop(0, n_chunks, body, 0)
        plsc.subcore_barrier()
        pltpu.sync_copy(
            acc_sh.at[pl.ds(s * n_per_tile, n_per_tile)],
            out_hbm.at[c].at[pl.ds(s * n_per_tile, n_per_tile)],
        )

    return k


def _matmul_call(x, w, deg_part, n_pad, rows):
    d_in = x.shape[1]
    d_out = w.shape[1]
    dh = d_out // 2
    nt = n_pad // rows

    def body(x_ref, w_ref, dp_ref, hs_ref):
        deg = 1.0 + dp_ref[0, :, 0] + dp_ref[1, :, 0]
        g = lax.rsqrt(deg)
        h = jnp.dot(x_ref[...], w_ref[...], preferred_element_type=jnp.float32)
        hs = h * g[:, None]
        hs_ref[0, :, :] = hs[:, :dh]
        hs_ref[1, :, :] = hs[:, dh:]

    return pl.pallas_call(
        body,
        grid=(nt,),
        in_specs=[
            pl.BlockSpec((rows, d_in), lambda i: (i, 0)),
            pl.BlockSpec((d_in, d_out), lambda i: (0, 0)),
            pl.BlockSpec((2, rows, L), lambda i: (0, i, 0)),
        ],
        out_specs=pl.BlockSpec((2, rows, dh), lambda i: (0, i, 0)),
        out_shape=jax.ShapeDtypeStruct((2, n_pad, dh), jnp.float32),
    )(x, w, deg_part)


def _combine_call(acc2, hs2, deg_part, b2, n, n_pad, rows):
    dh = acc2.shape[2]
    d_out = 2 * dh
    nt = n_pad // rows

    def body(acc_ref, hs_ref, dp_ref, b_ref, o_ref):
        deg = 1.0 + dp_ref[0, :, 0] + dp_ref[1, :, 0]
        g = lax.rsqrt(deg)
        lo = g[:, None] * (acc_ref[0] + hs_ref[0])
        hi = g[:, None] * (acc_ref[1] + hs_ref[1])
        o_ref[...] = jnp.concatenate([lo, hi], axis=1) + b_ref[...]

    return pl.pallas_call(
        body,
        grid=(nt,),
        in_specs=[
            pl.BlockSpec((2, rows, dh), lambda i: (0, i, 0)),
            pl.BlockSpec((2, rows, dh), lambda i: (0, i, 0)),
            pl.BlockSpec((2, rows, L), lambda i: (0, i, 0)),
            pl.BlockSpec((1, d_out), lambda i: (0, 0)),
        ],
        out_specs=pl.BlockSpec((rows, d_out), lambda i: (i, 0)),
        out_shape=jax.ShapeDtypeStruct((n, d_out), jnp.float32),
    )(acc2, hs2, deg_part, b2)


def kernel(x, edge_index, W, b):
    n, d_in = x.shape
    d_out = W.shape[1]
    e = edge_index.shape[1]
    dh = d_out // 2

    n_pad = -(-(n + 1) // 512) * 512
    rows = n_pad // 8

    src = edge_index[0].astype(jnp.int32)
    dst = edge_index[1].astype(jnp.int32)
    egrp = NC * NS * CHUNK
    e_pad = -(-e // egrp) * egrp
    if e_pad != e:
        fillv = jnp.full((e_pad - e,), n, jnp.int32)
        src = jnp.concatenate([src, fillv])
        dst = jnp.concatenate([dst, fillv])

    deg_part = _deg_kernel(n_pad, e_pad)(dst)
    hs2 = _matmul_call(x, W, deg_part, n_pad, rows)
    acc2 = _edge_kernel(n_pad, e_pad, dh)(src, dst, hs2)
    out = _combine_call(acc2, hs2, deg_part, b.reshape(1, -1), n, n_pad, rows)
    return out
